# baseline (speedup 1.0000x reference)
import functools

import jax
import jax.numpy as jnp
from jax import lax
from jax.experimental import pallas as pl
from jax.experimental.pallas import tpu as pltpu
from jax.experimental.pallas import tpu_sc as plsc

EMB_D = 256
N_CODES = 8192
N_TOK = 8192
COMMIT = 0.25

TOK_TILE = 256
CODE_CHUNK = 512
N_TOK_TILES = N_TOK // TOK_TILE
N_CODE_CHUNKS = N_CODES // CODE_CHUNK

SC_NC = 2
SC_NS = 16
SC_NW = SC_NC * SC_NS
GATHER_CHUNK = 128


def _vq_body(x_ref, ew_ref, idx_ref, enc_ref, loss_ref, ent_ref, cnt_ref,
             esq_ref):
    i = pl.program_id(0)

    @pl.when(i == 0)
    def _():
        for j in range(N_CODE_CHUNKS):
            e = ew_ref[pl.ds(j * CODE_CHUNK, CODE_CHUNK), :]
            esq_ref[pl.ds(j * CODE_CHUNK, CODE_CHUNK)] = jnp.sum(e * e, axis=1)

    x = x_ref[0]
    xsq = jnp.sum(x * x, axis=0)
    run2d = jnp.full((CODE_CHUNK, TOK_TILE), jnp.inf, jnp.float32)
    chk2d = jnp.zeros((CODE_CHUNK, TOK_TILE), jnp.int32)
    for j in range(N_CODE_CHUNKS):
        e = ew_ref[pl.ds(j * CODE_CHUNK, CODE_CHUNK), :]
        esq = esq_ref[pl.ds(j * CODE_CHUNK, CODE_CHUNK)]
        m = lax.dot_general(e, x, (((1,), (0,)), ((), ())),
                            preferred_element_type=jnp.float32)
        dist = (xsq[None, :] + esq[:, None]) - 2.0 * m
        better = dist < run2d
        run2d = jnp.where(better, dist, run2d)
        chk2d = jnp.where(better, j, chk2d)
    run_min = jnp.min(run2d, axis=0)
    slot = lax.broadcasted_iota(jnp.int32, (CODE_CHUNK, TOK_TILE), 0)
    code2d = chk2d * CODE_CHUNK + slot
    cand = jnp.where(run2d == run_min[None, :], code2d, N_CODES)
    run_arg = jnp.min(cand, axis=0)
    idx_ref[0, 0, :] = run_arg

    arg_col = jnp.transpose(jnp.reshape(run_arg, (1, TOK_TILE)))
    for j in range(N_CODE_CHUNKS):
        iota = lax.broadcasted_iota(jnp.int32, (TOK_TILE, CODE_CHUNK), 1)
        enc = (iota + j * CODE_CHUNK == arg_col).astype(jnp.float32)
        enc_ref[:, pl.ds(j * CODE_CHUNK, CODE_CHUNK)] = enc
        colsum = jnp.sum(enc, axis=0)
        sl = pl.ds(j * CODE_CHUNK, CODE_CHUNK)

        @pl.when(i == 0)
        def _():
            cnt_ref[sl] = colsum

        @pl.when(i > 0)
        def _():
            cnt_ref[sl] = cnt_ref[sl] + colsum

    partial = jnp.reshape(jnp.sum(run_min), (1, 1))

    @pl.when(i == 0)
    def _():
        loss_ref[...] = jnp.zeros((1, 1), jnp.float32)
        ent_ref[...] = jnp.zeros((1, 1), jnp.float32)

    loss_ref[...] += partial

    @pl.when(i == N_TOK_TILES - 1)
    def _():
        loss_ref[...] = loss_ref[...] * (COMMIT / (N_TOK * EMB_D))
        p = cnt_ref[...] * (1.0 / N_TOK)
        ent_ref[...] = jnp.reshape(-jnp.sum(p * jnp.log(p + 1e-10)), (1, 1))


@functools.lru_cache(maxsize=1)
def _make_sc_gather():
    mesh = plsc.VectorSubcoreMesh(core_axis_name="c", subcore_axis_name="s")

    n_chunks = (N_TOK // SC_NW) // GATHER_CHUNK

    b_per_w = N_TOK // SC_NW

    @functools.partial(
        pl.kernel,
        mesh=mesh,
        out_type=jax.ShapeDtypeStruct((8 * EMB_D * 1024,), jnp.float32),
        scratch_types=[
            pltpu.VMEM((GATHER_CHUNK,), jnp.int32),
            pltpu.VMEM((GATHER_CHUNK, EMB_D), jnp.float32),
            pltpu.VMEM((EMB_D * b_per_w,), jnp.float32),
            pltpu.SemaphoreType.DMA,
            pltpu.SemaphoreType.DMA,
        ],
        compiler_params=pltpu.CompilerParams(needs_layout_passes=False),
    )
    def _sc_gather(table_hbm, idx_hbm, out_hbm, idx_v, rows_v, rows_t,
                   gsem, wsem):
        wid = lax.axis_index("s") * SC_NC + lax.axis_index("c")
        base = wid * b_per_w
        b = base // 1024
        hw0 = base % 1024
        iota = lax.broadcasted_iota(jnp.int32, (16,), 0)
        for j in range(n_chunks):
            pltpu.sync_copy(idx_hbm.at[pl.ds(base + j * GATHER_CHUNK,
                                             GATHER_CHUNK)], idx_v)
            pltpu.async_copy(table_hbm.at[idx_v], rows_v, gsem).wait()

            def _tp(r, carry):
                t = j * GATHER_CHUNK + r
                for g in range(EMB_D // 16):
                    val = rows_v[r, pl.ds(g * 16, 16)]
                    plsc.store_scatter(rows_t, [(g * 16 + iota) * b_per_w + t],
                                       val)
                return carry

            lax.fori_loop(0, GATHER_CHUNK, _tp, 0)

        out_base = b * EMB_D * 1024 + hw0

        def _fire(d, carry):
            pltpu.async_copy(
                rows_t.at[pl.ds(d * b_per_w, b_per_w)],
                out_hbm.at[pl.ds(out_base + d * 1024, b_per_w)], wsem)
            return carry

        lax.fori_loop(0, EMB_D, _fire, 0)
        pltpu.make_async_copy(
            out_hbm.at[pl.ds(0, EMB_D * b_per_w)], rows_t, wsem).wait()

    return _sc_gather


def kernel(inputs, embedding_weight):
    x3 = inputs.reshape(8, EMB_D, 1024)

    idx3, enc, loss11, ent11 = pl.pallas_call(
        _vq_body,
        grid=(N_TOK_TILES,),
        in_specs=[
            pl.BlockSpec((1, EMB_D, TOK_TILE),
                         lambda i: (i // (1024 // TOK_TILE), 0,
                                    i % (1024 // TOK_TILE))),
            pl.BlockSpec((N_CODES, EMB_D), lambda i: (0, 0)),
        ],
        out_specs=[
            pl.BlockSpec((1, 1, TOK_TILE), lambda i: (i, 0, 0)),
            pl.BlockSpec((TOK_TILE, N_CODES), lambda i: (i, 0)),
            pl.BlockSpec((1, 1), lambda i: (0, 0)),
            pl.BlockSpec((1, 1), lambda i: (0, 0)),
        ],
        out_shape=[
            jax.ShapeDtypeStruct((N_TOK_TILES, 1, TOK_TILE), jnp.int32),
            jax.ShapeDtypeStruct((N_TOK, N_CODES), jnp.float32),
            jax.ShapeDtypeStruct((1, 1), jnp.float32),
            jax.ShapeDtypeStruct((1, 1), jnp.float32),
        ],
        scratch_shapes=[pltpu.VMEM((N_CODES,), jnp.float32),
                        pltpu.VMEM((N_CODES,), jnp.float32)],
    )(x3, embedding_weight)

    q1d = _make_sc_gather()(embedding_weight, idx3.reshape(N_TOK))
    quantized = q1d.reshape(8, EMB_D, 32, 32)
    return (quantized, loss11[0, 0], ent11[0, 0], enc)


# speedup vs baseline: 1.2709x; 1.2709x over previous
import functools

import jax
import jax.numpy as jnp
from jax import lax
from jax.experimental import pallas as pl
from jax.experimental.pallas import tpu as pltpu
from jax.experimental.pallas import tpu_sc as plsc

EMB_D = 256
N_CODES = 8192
N_TOK = 8192
COMMIT = 0.25

TOK_TILE = 256
CODE_CHUNK = 512
N_CODE_CHUNKS = N_CODES // CODE_CHUNK
HALF_TOK = N_TOK // 2
HALF_TILES = HALF_TOK // TOK_TILE

SC_NC = 2
SC_NS = 16
SC_NW = SC_NC * SC_NS
GATHER_CHUNK = 128


def _common_tile(x, ew_ref, esq_ref, idx_ref):
    xsq = jnp.sum(x * x, axis=0)
    run2d = jnp.full((CODE_CHUNK, TOK_TILE), jnp.inf, jnp.float32)
    chk2d = jnp.zeros((CODE_CHUNK, TOK_TILE), jnp.int32)
    for j in range(N_CODE_CHUNKS):
        e = ew_ref[pl.ds(j * CODE_CHUNK, CODE_CHUNK), :]
        esq = esq_ref[0, 0, pl.ds(j * CODE_CHUNK, CODE_CHUNK)]
        m = lax.dot_general(e, x, (((1,), (0,)), ((), ())),
                            preferred_element_type=jnp.float32)
        dist = (xsq[None, :] + esq[:, None]) - 2.0 * m
        better = dist < run2d
        run2d = jnp.where(better, dist, run2d)
        chk2d = jnp.where(better, j, chk2d)
    run_min = jnp.min(run2d, axis=0)
    slot = lax.broadcasted_iota(jnp.int32, (CODE_CHUNK, TOK_TILE), 0)
    code2d = chk2d * CODE_CHUNK + slot
    cand = jnp.where(run2d == run_min[None, :], code2d, N_CODES)
    run_arg = jnp.min(cand, axis=0)
    idx_ref[0, 0, :] = run_arg
    return run_min, run_arg


def _onehot_tile(i, run_arg, enc_ref, cnt_init, cnt_ref):
    arg_col = jnp.transpose(jnp.reshape(run_arg, (1, TOK_TILE)))
    for j in range(N_CODE_CHUNKS):
        iota = lax.broadcasted_iota(jnp.int32, (TOK_TILE, CODE_CHUNK), 1)
        enc = (iota + j * CODE_CHUNK == arg_col).astype(jnp.float32)
        enc_ref[:, pl.ds(j * CODE_CHUNK, CODE_CHUNK)] = enc
        colsum = jnp.sum(enc, axis=0)
        sl = pl.ds(j * CODE_CHUNK, CODE_CHUNK)

        @pl.when(i == 0)
        def _():
            cnt_ref[sl] = cnt_init(sl) + colsum

        @pl.when(i > 0)
        def _():
            cnt_ref[sl] = cnt_ref[sl] + colsum


def _vq_body_h1(x_ref, ew_ref, idx_ref, enc_ref, loss_ref, esq_ref, cnt_ref):
    i = pl.program_id(0)

    @pl.when(i == 0)
    def _():
        for j in range(N_CODE_CHUNKS):
            e = ew_ref[pl.ds(j * CODE_CHUNK, CODE_CHUNK), :]
            esq_ref[0, 0, pl.ds(j * CODE_CHUNK, CODE_CHUNK)] = (
                jnp.sum(e * e, axis=1))

    run_min, run_arg = _common_tile(x_ref[0], ew_ref, esq_ref, idx_ref)
    _onehot_tile(i, run_arg,
                 enc_ref,
                 lambda sl: jnp.zeros((CODE_CHUNK,), jnp.float32),
                 cnt_ref.at[0, 0])

    partial = jnp.reshape(jnp.sum(run_min), (1, 1))

    @pl.when(i == 0)
    def _():
        loss_ref[...] = jnp.zeros((1, 1), jnp.float32)

    loss_ref[...] += partial


def _vq_body_h2(x_ref, ew_ref, esq_ref, cnt1_ref, loss1_ref, enc_any,
                idx_ref, enc_ref, loss_ref, ent_ref, cnt_ref):
    del enc_any
    i = pl.program_id(0)
    run_min, run_arg = _common_tile(x_ref[0], ew_ref, esq_ref, idx_ref)
    _onehot_tile(i, run_arg,
                 enc_ref,
                 lambda sl: cnt1_ref[0, 0, sl],
                 cnt_ref)

    partial = jnp.reshape(jnp.sum(run_min), (1, 1))

    @pl.when(i == 0)
    def _():
        loss_ref[...] = loss1_ref[...]

    loss_ref[...] += partial

    @pl.when(i == HALF_TILES - 1)
    def _():
        loss_ref[...] = loss_ref[...] * (COMMIT / (N_TOK * EMB_D))
        p = cnt_ref[...] * (1.0 / N_TOK)
        ent_ref[...] = jnp.reshape(-jnp.sum(p * jnp.log(p + 1e-10)), (1, 1))


@functools.lru_cache(maxsize=2)
def _make_sc_gather(n_tok):
    mesh = plsc.VectorSubcoreMesh(core_axis_name="c", subcore_axis_name="s")
    b_per_w = n_tok // SC_NW
    n_chunks = b_per_w // GATHER_CHUNK

    @functools.partial(
        pl.kernel,
        mesh=mesh,
        out_type=jax.ShapeDtypeStruct((n_tok, EMB_D), jnp.float32),
        scratch_types=[
            pltpu.VMEM((GATHER_CHUNK,), jnp.int32),
            pltpu.VMEM((GATHER_CHUNK, EMB_D), jnp.float32),
            pltpu.SemaphoreType.DMA,
        ],
    )
    def _sc_gather(table_hbm, idx_hbm, out_hbm, idx_v, rows_v, sem):
        wid = lax.axis_index("s") * SC_NC + lax.axis_index("c")
        base = wid * b_per_w
        for j in range(n_chunks):
            off = base + j * GATHER_CHUNK
            pltpu.sync_copy(idx_hbm.at[pl.ds(off, GATHER_CHUNK)], idx_v)
            pltpu.async_copy(table_hbm.at[idx_v], rows_v, sem).wait()
            pltpu.sync_copy(rows_v, out_hbm.at[pl.ds(off, GATHER_CHUNK)])

    return _sc_gather


def kernel(inputs, embedding_weight):
    x3 = inputs.reshape(8, EMB_D, 1024)
    tiles_per_img = 1024 // TOK_TILE

    idx1, enc1, loss1, esq, cnt1 = pl.pallas_call(
        _vq_body_h1,
        grid=(HALF_TILES,),
        in_specs=[
            pl.BlockSpec((1, EMB_D, TOK_TILE),
                         lambda i: (i // tiles_per_img, 0, i % tiles_per_img)),
            pl.BlockSpec((N_CODES, EMB_D), lambda i: (0, 0)),
        ],
        out_specs=[
            pl.BlockSpec((1, 1, TOK_TILE), lambda i: (i, 0, 0)),
            pl.BlockSpec((TOK_TILE, N_CODES), lambda i: (i, 0)),
            pl.BlockSpec((1, 1), lambda i: (0, 0)),
            pl.BlockSpec((1, 1, N_CODES), lambda i: (0, 0, 0)),
            pl.BlockSpec((1, 1, N_CODES), lambda i: (0, 0, 0)),
        ],
        out_shape=[
            jax.ShapeDtypeStruct((HALF_TILES, 1, TOK_TILE), jnp.int32),
            jax.ShapeDtypeStruct((N_TOK, N_CODES), jnp.float32),
            jax.ShapeDtypeStruct((1, 1), jnp.float32),
            jax.ShapeDtypeStruct((1, 1, N_CODES), jnp.float32),
            jax.ShapeDtypeStruct((1, 1, N_CODES), jnp.float32),
        ],
    )(x3, embedding_weight)

    idx2, enc, loss11, ent11 = pl.pallas_call(
        _vq_body_h2,
        grid=(HALF_TILES,),
        in_specs=[
            pl.BlockSpec((1, EMB_D, TOK_TILE),
                         lambda i: (4 + i // tiles_per_img, 0,
                                    i % tiles_per_img)),
            pl.BlockSpec((N_CODES, EMB_D), lambda i: (0, 0)),
            pl.BlockSpec((1, 1, N_CODES), lambda i: (0, 0, 0)),
            pl.BlockSpec((1, 1, N_CODES), lambda i: (0, 0, 0)),
            pl.BlockSpec((1, 1), lambda i: (0, 0)),
            pl.BlockSpec(memory_space=pl.ANY),
        ],
        out_specs=[
            pl.BlockSpec((1, 1, TOK_TILE), lambda i: (i, 0, 0)),
            pl.BlockSpec((TOK_TILE, N_CODES), lambda i: (HALF_TILES + i, 0)),
            pl.BlockSpec((1, 1), lambda i: (0, 0)),
            pl.BlockSpec((1, 1), lambda i: (0, 0)),
        ],
        out_shape=[
            jax.ShapeDtypeStruct((HALF_TILES, 1, TOK_TILE), jnp.int32),
            jax.ShapeDtypeStruct((N_TOK, N_CODES), jnp.float32),
            jax.ShapeDtypeStruct((1, 1), jnp.float32),
            jax.ShapeDtypeStruct((1, 1), jnp.float32),
        ],
        input_output_aliases={5: 1},
        scratch_shapes=[pltpu.VMEM((N_CODES,), jnp.float32)],
    )(x3, embedding_weight, esq, cnt1, loss1, enc1)

    gather = _make_sc_gather(HALF_TOK)
    q1 = gather(embedding_weight, idx1.reshape(HALF_TOK))
    q2 = gather(embedding_weight, idx2.reshape(HALF_TOK))
    t1 = jnp.transpose(q1.reshape(4, 32, 32, EMB_D), (0, 3, 1, 2))
    t2 = jnp.transpose(q2.reshape(4, 32, 32, EMB_D), (0, 3, 1, 2))
    quantized = jnp.concatenate([t1, t2], axis=0)
    return (quantized, loss11[0, 0], ent11[0, 0], enc)


# speedup vs baseline: 1.3880x; 1.0922x over previous
import functools

import jax
import jax.numpy as jnp
from jax import lax
from jax.experimental import pallas as pl
from jax.experimental.pallas import tpu as pltpu
from jax.experimental.pallas import tpu_sc as plsc

EMB_D = 256
N_CODES = 8192
N_TOK = 8192
COMMIT = 0.25

TOK_TILE = 256
CODE_CHUNK = 512
N_TOK_TILES = N_TOK // TOK_TILE
N_CODE_CHUNKS = N_CODES // CODE_CHUNK

SC_NC = 2
SC_NS = 16
SC_NW = SC_NC * SC_NS
GATHER_CHUNK = 128


def _vq_body(x_ref, ew_ref, idx_ref, enc_ref, loss_ref, ent_ref, cnt_ref,
             esq_ref):
    i = pl.program_id(0)

    @pl.when(i == 0)
    def _():
        for j in range(N_CODE_CHUNKS):
            e = ew_ref[pl.ds(j * CODE_CHUNK, CODE_CHUNK), :]
            esq_ref[pl.ds(j * CODE_CHUNK, CODE_CHUNK)] = jnp.sum(e * e, axis=1)

    x = x_ref[0]
    xsq = jnp.sum(x * x, axis=0)
    run2d = jnp.full((CODE_CHUNK, TOK_TILE), jnp.inf, jnp.float32)
    chk2d = jnp.zeros((CODE_CHUNK, TOK_TILE), jnp.int32)
    for j in range(N_CODE_CHUNKS):
        e = ew_ref[pl.ds(j * CODE_CHUNK, CODE_CHUNK), :]
        esq = esq_ref[pl.ds(j * CODE_CHUNK, CODE_CHUNK)]
        m = lax.dot_general(e, x, (((1,), (0,)), ((), ())),
                            preferred_element_type=jnp.float32)
        dist = (xsq[None, :] + esq[:, None]) - 2.0 * m
        better = dist < run2d
        run2d = jnp.where(better, dist, run2d)
        chk2d = jnp.where(better, j, chk2d)
    run_min = jnp.min(run2d, axis=0)
    slot = lax.broadcasted_iota(jnp.int32, (CODE_CHUNK, TOK_TILE), 0)
    code2d = chk2d * CODE_CHUNK + slot
    cand = jnp.where(run2d == run_min[None, :], code2d, N_CODES)
    run_arg = jnp.min(cand, axis=0)
    idx_ref[0, 0, :] = run_arg

    arg_col = jnp.transpose(jnp.reshape(run_arg, (1, TOK_TILE)))
    for j in range(N_CODE_CHUNKS):
        iota = lax.broadcasted_iota(jnp.int32, (TOK_TILE, CODE_CHUNK), 1)
        enc = (iota + j * CODE_CHUNK == arg_col).astype(jnp.float32)
        enc_ref[:, pl.ds(j * CODE_CHUNK, CODE_CHUNK)] = enc
        colsum = jnp.sum(enc, axis=0)
        sl = pl.ds(j * CODE_CHUNK, CODE_CHUNK)

        @pl.when(i == 0)
        def _():
            cnt_ref[sl] = colsum

        @pl.when(i > 0)
        def _():
            cnt_ref[sl] = cnt_ref[sl] + colsum

    partial = jnp.reshape(jnp.sum(run_min), (1, 1))

    @pl.when(i == 0)
    def _():
        loss_ref[...] = jnp.zeros((1, 1), jnp.float32)
        ent_ref[...] = jnp.zeros((1, 1), jnp.float32)

    loss_ref[...] += partial

    @pl.when(i == N_TOK_TILES - 1)
    def _():
        loss_ref[...] = loss_ref[...] * (COMMIT / (N_TOK * EMB_D))
        p = cnt_ref[...] * (1.0 / N_TOK)
        ent_ref[...] = jnp.reshape(-jnp.sum(p * jnp.log(p + 1e-10)), (1, 1))


@functools.lru_cache(maxsize=1)
def _make_sc_gather():
    mesh = plsc.VectorSubcoreMesh(core_axis_name="c", subcore_axis_name="s")
    b_per_w = N_TOK // SC_NW
    n_chunks = b_per_w // GATHER_CHUNK

    @functools.partial(
        pl.kernel,
        mesh=mesh,
        out_type=jax.ShapeDtypeStruct((N_TOK, EMB_D), jnp.float32),
        scratch_types=[
            pltpu.VMEM((GATHER_CHUNK,), jnp.int32),
            pltpu.VMEM((GATHER_CHUNK, EMB_D), jnp.float32),
            pltpu.SemaphoreType.DMA,
        ],
    )
    def _sc_gather(table_hbm, idx_hbm, out_hbm, idx_v, rows_v, sem):
        wid = lax.axis_index("s") * SC_NC + lax.axis_index("c")
        base = wid * b_per_w
        for j in range(n_chunks):
            off = base + j * GATHER_CHUNK
            pltpu.sync_copy(idx_hbm.at[pl.ds(off, GATHER_CHUNK)], idx_v)
            pltpu.async_copy(table_hbm.at[idx_v], rows_v, sem).wait()
            pltpu.sync_copy(rows_v, out_hbm.at[pl.ds(off, GATHER_CHUNK)])

    return _sc_gather


def kernel(inputs, embedding_weight):
    x3 = inputs.reshape(8, EMB_D, 1024)
    tiles_per_img = 1024 // TOK_TILE

    idx3, enc, loss11, ent11 = pl.pallas_call(
        _vq_body,
        grid=(N_TOK_TILES,),
        in_specs=[
            pl.BlockSpec((1, EMB_D, TOK_TILE),
                         lambda i: (i // tiles_per_img, 0, i % tiles_per_img)),
            pl.BlockSpec((N_CODES, EMB_D), lambda i: (0, 0)),
        ],
        out_specs=[
            pl.BlockSpec((1, 1, TOK_TILE), lambda i: (i, 0, 0)),
            pl.BlockSpec((TOK_TILE, N_CODES), lambda i: (i, 0)),
            pl.BlockSpec((1, 1), lambda i: (0, 0)),
            pl.BlockSpec((1, 1), lambda i: (0, 0)),
        ],
        out_shape=[
            jax.ShapeDtypeStruct((N_TOK_TILES, 1, TOK_TILE), jnp.int32),
            jax.ShapeDtypeStruct((N_TOK, N_CODES), jnp.float32),
            jax.ShapeDtypeStruct((1, 1), jnp.float32),
            jax.ShapeDtypeStruct((1, 1), jnp.float32),
        ],
        scratch_shapes=[pltpu.VMEM((N_CODES,), jnp.float32),
                        pltpu.VMEM((N_CODES,), jnp.float32)],
    )(x3, embedding_weight)

    q_flat = _make_sc_gather()(embedding_weight, idx3.reshape(N_TOK))
    quantized = jnp.transpose(q_flat.reshape(8, 32, 32, EMB_D), (0, 3, 1, 2))
    return (quantized, loss11[0, 0], ent11[0, 0], enc)


# speedup vs baseline: 1.4041x; 1.0116x over previous
import functools

import jax
import jax.numpy as jnp
from jax import lax
from jax.experimental import pallas as pl
from jax.experimental.pallas import tpu as pltpu
from jax.experimental.pallas import tpu_sc as plsc

EMB_D = 256
N_CODES = 8192
N_TOK = 8192
COMMIT = 0.25

TOK_TILE = 512
CODE_CHUNK = 512
N_TOK_TILES = N_TOK // TOK_TILE
N_CODE_CHUNKS = N_CODES // CODE_CHUNK

SC_NC = 2
SC_NS = 16
SC_NW = SC_NC * SC_NS
GATHER_CHUNK = 128


def _vq_body(x_ref, ew_ref, idx_ref, enc_ref, loss_ref, ent_ref, cnt_ref,
             esq_ref):
    i = pl.program_id(0)

    @pl.when(i == 0)
    def _():
        for j in range(N_CODE_CHUNKS):
            e = ew_ref[pl.ds(j * CODE_CHUNK, CODE_CHUNK), :]
            esq_ref[pl.ds(j * CODE_CHUNK, CODE_CHUNK)] = jnp.sum(e * e, axis=1)

    x = x_ref[0]
    xsq = jnp.sum(x * x, axis=0)
    run2d = jnp.full((CODE_CHUNK, TOK_TILE), jnp.inf, jnp.float32)
    chk2d = jnp.zeros((CODE_CHUNK, TOK_TILE), jnp.int32)
    for j in range(N_CODE_CHUNKS):
        e = ew_ref[pl.ds(j * CODE_CHUNK, CODE_CHUNK), :]
        esq = esq_ref[pl.ds(j * CODE_CHUNK, CODE_CHUNK)]
        m = lax.dot_general(e, x, (((1,), (0,)), ((), ())),
                            preferred_element_type=jnp.float32)
        dist = (xsq[None, :] + esq[:, None]) - 2.0 * m
        better = dist < run2d
        run2d = jnp.where(better, dist, run2d)
        chk2d = jnp.where(better, j, chk2d)
    run_min = jnp.min(run2d, axis=0)
    slot = lax.broadcasted_iota(jnp.int32, (CODE_CHUNK, TOK_TILE), 0)
    code2d = chk2d * CODE_CHUNK + slot
    cand = jnp.where(run2d == run_min[None, :], code2d, N_CODES)
    run_arg = jnp.min(cand, axis=0)
    idx_ref[0, 0, :] = run_arg

    arg_col = jnp.transpose(jnp.reshape(run_arg, (1, TOK_TILE)))
    for j in range(N_CODE_CHUNKS):
        iota = lax.broadcasted_iota(jnp.int32, (TOK_TILE, CODE_CHUNK), 1)
        enc = (iota + j * CODE_CHUNK == arg_col).astype(jnp.float32)
        enc_ref[:, pl.ds(j * CODE_CHUNK, CODE_CHUNK)] = enc
        colsum = jnp.sum(enc, axis=0)
        sl = pl.ds(j * CODE_CHUNK, CODE_CHUNK)

        @pl.when(i == 0)
        def _():
            cnt_ref[sl] = colsum

        @pl.when(i > 0)
        def _():
            cnt_ref[sl] = cnt_ref[sl] + colsum

    partial = jnp.reshape(jnp.sum(run_min), (1, 1))

    @pl.when(i == 0)
    def _():
        loss_ref[...] = jnp.zeros((1, 1), jnp.float32)
        ent_ref[...] = jnp.zeros((1, 1), jnp.float32)

    loss_ref[...] += partial

    @pl.when(i == N_TOK_TILES - 1)
    def _():
        loss_ref[...] = loss_ref[...] * (COMMIT / (N_TOK * EMB_D))
        p = cnt_ref[...] * (1.0 / N_TOK)
        ent_ref[...] = jnp.reshape(-jnp.sum(p * jnp.log(p + 1e-10)), (1, 1))


@functools.lru_cache(maxsize=1)
def _make_sc_gather():
    mesh = plsc.VectorSubcoreMesh(core_axis_name="c", subcore_axis_name="s")
    b_per_w = N_TOK // SC_NW
    n_chunks = b_per_w // GATHER_CHUNK

    @functools.partial(
        pl.kernel,
        mesh=mesh,
        out_type=jax.ShapeDtypeStruct((N_TOK, EMB_D), jnp.float32),
        scratch_types=[
            pltpu.VMEM((GATHER_CHUNK,), jnp.int32),
            pltpu.VMEM((GATHER_CHUNK, EMB_D), jnp.float32),
            pltpu.SemaphoreType.DMA,
        ],
    )
    def _sc_gather(table_hbm, idx_hbm, out_hbm, idx_v, rows_v, sem):
        wid = lax.axis_index("s") * SC_NC + lax.axis_index("c")
        base = wid * b_per_w
        for j in range(n_chunks):
            off = base + j * GATHER_CHUNK
            pltpu.sync_copy(idx_hbm.at[pl.ds(off, GATHER_CHUNK)], idx_v)
            pltpu.async_copy(table_hbm.at[idx_v], rows_v, sem).wait()
            pltpu.sync_copy(rows_v, out_hbm.at[pl.ds(off, GATHER_CHUNK)])

    return _sc_gather


def kernel(inputs, embedding_weight):
    x3 = inputs.reshape(8, EMB_D, 1024)
    tiles_per_img = 1024 // TOK_TILE

    idx3, enc, loss11, ent11 = pl.pallas_call(
        _vq_body,
        grid=(N_TOK_TILES,),
        in_specs=[
            pl.BlockSpec((1, EMB_D, TOK_TILE),
                         lambda i: (i // tiles_per_img, 0, i % tiles_per_img)),
            pl.BlockSpec((N_CODES, EMB_D), lambda i: (0, 0)),
        ],
        out_specs=[
            pl.BlockSpec((1, 1, TOK_TILE), lambda i: (i, 0, 0)),
            pl.BlockSpec((TOK_TILE, N_CODES), lambda i: (i, 0)),
            pl.BlockSpec((1, 1), lambda i: (0, 0)),
            pl.BlockSpec((1, 1), lambda i: (0, 0)),
        ],
        out_shape=[
            jax.ShapeDtypeStruct((N_TOK_TILES, 1, TOK_TILE), jnp.int32),
            jax.ShapeDtypeStruct((N_TOK, N_CODES), jnp.float32),
            jax.ShapeDtypeStruct((1, 1), jnp.float32),
            jax.ShapeDtypeStruct((1, 1), jnp.float32),
        ],
        scratch_shapes=[pltpu.VMEM((N_CODES,), jnp.float32),
                        pltpu.VMEM((N_CODES,), jnp.float32)],
    )(x3, embedding_weight)

    q_flat = _make_sc_gather()(embedding_weight, idx3.reshape(N_TOK))
    quantized = jnp.transpose(q_flat.reshape(8, 32, 32, EMB_D), (0, 3, 1, 2))
    return (quantized, loss11[0, 0], ent11[0, 0], enc)
